# baseline (speedup 1.0000x reference)
import functools

import jax
import jax.numpy as jnp
from jax import lax
from jax.experimental import pallas as pl
from jax.experimental.pallas import tpu as pltpu
from jax.experimental.pallas import tpu_sc as plsc

D = 64
B = 4096
H = 200
NC, NS = 2, 16
NW = NC * NS
HT = H // 8
BT = B // 128


@functools.partial(
    pl.kernel,
    mesh=plsc.VectorSubcoreMesh(core_axis_name="c", subcore_axis_name="s"),
    out_type=jax.ShapeDtypeStruct((H, BT, 128, D), jnp.float32),
    scratch_types=[
        pltpu.VMEM((HT, 8, 128), jnp.int32),
        pltpu.VMEM((128, D), jnp.float32),
        pltpu.VMEM((128, D), jnp.float32),
        pltpu.SemaphoreType.DMA,
        pltpu.SemaphoreType.DMA,
        pltpu.SemaphoreType.DMA,
        pltpu.SemaphoreType.DMA,
    ],
    compiler_params=pltpu.CompilerParams(
        needs_layout_passes=False, use_tc_tiling_on_sc=False),
)
def _sc_gather(x4_hbm, table_hbm, out_hbm,
               idxs, rows_a, rows_b, gsem_a, gsem_b, wsem_a, wsem_b):
    w = lax.axis_index("s") * NC + lax.axis_index("c")

    for ht in range(HT):
        pltpu.sync_copy(x4_hbm.at[ht, w], idxs.at[ht])

    bufs = ((rows_a, gsem_a, wsem_a), (rows_b, gsem_b, wsem_b))

    def fire_gather(t, par):
        rows, gsem, _ = bufs[par]
        pltpu.async_copy(table_hbm.at[idxs.at[lax.div(t, 8), lax.rem(t, 8)]],
                         rows, gsem)

    def wait_gather(t, par):
        rows, gsem, _ = bufs[par]
        pltpu.make_async_copy(table_hbm.at[idxs.at[lax.div(t, 8), lax.rem(t, 8)]],
                              rows, gsem).wait()

    def fire_writeout(t, par):
        rows, _, wsem = bufs[par]
        pltpu.async_copy(rows, out_hbm.at[t, w], wsem)

    def wait_writeout(t, par):
        rows, _, wsem = bufs[par]
        pltpu.make_async_copy(rows, out_hbm.at[t, w], wsem).wait()

    fire_gather(0, 0)

    def pair_body(i, carry):
        for par in range(2):
            t = i * 2 + par

            wait_gather(t, par)

            @pl.when(t >= 1)
            def _():
                wait_writeout(t - 1, 1 - par)

            @pl.when(t + 1 < H)
            def _():
                fire_gather(t + 1, 1 - par)

            fire_writeout(t, par)
        return carry

    lax.fori_loop(0, H // 2, pair_body, 0)
    wait_writeout(H - 1, 1)


def _tc_ln_kernel(x_ref, g_ref, b_ref, o_ref):
    x = x_ref[...].reshape(BT * 64, 2, D)
    s = jnp.sum(x, axis=-1, keepdims=True)
    ss = jnp.sum(x * x, axis=-1, keepdims=True)
    mean = s * (1.0 / D)
    var = ss * (1.0 / D) - mean * mean
    rstd = lax.rsqrt(var + 1e-5)
    g = g_ref[0].reshape(1, 1, D)
    b = b_ref[0].reshape(1, 1, D)
    o_ref[...] = ((x - mean) * rstd * g + b).reshape(1, BT, 64, 128)


def _tc_ln(scr, gamma, beta):
    return pl.pallas_call(
        _tc_ln_kernel,
        grid=(H,),
        in_specs=[
            pl.BlockSpec((1, BT, 64, 128), lambda h: (h, 0, 0, 0)),
            pl.BlockSpec((1, D), lambda h: (0, 0)),
            pl.BlockSpec((1, D), lambda h: (0, 0)),
        ],
        out_specs=pl.BlockSpec((1, BT, 64, 128), lambda h: (h, 0, 0, 0)),
        out_shape=jax.ShapeDtypeStruct((H, BT, 64, 128), jnp.float32),
    )(scr, gamma, beta)


def kernel(x, table, gamma, beta):
    x4 = x.astype(jnp.int32).T.reshape(HT, 8, BT, 128).transpose(0, 2, 1, 3)
    scr = _sc_gather(x4, table)
    packed = _tc_ln(scr.reshape(H, BT, 64, 128),
                    gamma.reshape(1, D), beta.reshape(1, D))
    out = (packed.reshape(H, BT, 64, 2, D)
           .transpose(1, 2, 3, 0, 4)
           .reshape(B, H, D))
    return out


# speedup vs baseline: 2.1198x; 2.1198x over previous
import functools

import jax
import jax.numpy as jnp
from jax import lax
from jax.experimental import pallas as pl
from jax.experimental.pallas import tpu as pltpu
from jax.experimental.pallas import tpu_sc as plsc

D = 64
B = 4096
H = 200
NC, NS = 2, 16
NW = NC * NS
HT = H // 8
BT = B // 128


@functools.partial(
    pl.kernel,
    mesh=plsc.VectorSubcoreMesh(core_axis_name="c", subcore_axis_name="s"),
    out_type=jax.ShapeDtypeStruct((H, BT, 128, D), jnp.float32),
    scratch_types=[
        pltpu.VMEM((HT, 8, 128), jnp.int32),
        pltpu.VMEM((128, D), jnp.float32),
        pltpu.VMEM((128, D), jnp.float32),
        pltpu.SemaphoreType.DMA,
        pltpu.SemaphoreType.DMA,
        pltpu.SemaphoreType.DMA,
        pltpu.SemaphoreType.DMA,
    ],
    compiler_params=pltpu.CompilerParams(
        needs_layout_passes=False, use_tc_tiling_on_sc=False),
)
def _sc_gather(x4_hbm, table_hbm, out_hbm,
               idxs, rows_a, rows_b, gsem_a, gsem_b, wsem_a, wsem_b):
    w = lax.axis_index("s") * NC + lax.axis_index("c")

    for ht in range(HT):
        pltpu.sync_copy(x4_hbm.at[ht, w], idxs.at[ht])

    bufs = ((rows_a, gsem_a, wsem_a), (rows_b, gsem_b, wsem_b))

    def fire_gather(t, par):
        rows, gsem, _ = bufs[par]
        pltpu.async_copy(table_hbm.at[idxs.at[lax.div(t, 8), lax.rem(t, 8)]],
                         rows, gsem)

    def wait_gather(t, par):
        rows, gsem, _ = bufs[par]
        pltpu.make_async_copy(table_hbm.at[idxs.at[lax.div(t, 8), lax.rem(t, 8)]],
                              rows, gsem).wait()

    def fire_writeout(t, par):
        rows, _, wsem = bufs[par]
        pltpu.async_copy(rows, out_hbm.at[t, w], wsem)

    def wait_writeout(t, par):
        rows, _, wsem = bufs[par]
        pltpu.make_async_copy(rows, out_hbm.at[t, w], wsem).wait()

    fire_gather(0, 0)

    def pair_body(i, carry):
        for par in range(2):
            t = i * 2 + par

            wait_gather(t, par)

            @pl.when(t >= 1)
            def _():
                wait_writeout(t - 1, 1 - par)

            @pl.when(t + 1 < H)
            def _():
                fire_gather(t + 1, 1 - par)

            fire_writeout(t, par)
        return carry

    lax.fori_loop(0, H // 2, pair_body, 0)
    wait_writeout(H - 1, 1)


def _tc_ln_kernel(x_ref, g_ref, b_ref, o_ref):
    x = x_ref[0]
    s = jnp.sum(x, axis=-1, keepdims=True)
    ss = jnp.sum(x * x, axis=-1, keepdims=True)
    mean = s * (1.0 / D)
    var = ss * (1.0 / D) - mean * mean
    rstd = lax.rsqrt(var + 1e-5)
    g = g_ref[0].reshape(1, 1, D)
    b = b_ref[0].reshape(1, 1, D)
    y = (x - mean) * rstd * g + b
    yt = y.transpose(0, 2, 1)
    o_ref[0] = yt.reshape(BT, 8, 8, 128).transpose(1, 0, 2, 3)


def _tc_ln(scr, gamma, beta):
    return pl.pallas_call(
        _tc_ln_kernel,
        grid=(H,),
        in_specs=[
            pl.BlockSpec((1, BT, 128, D), lambda h: (h, 0, 0, 0)),
            pl.BlockSpec((1, D), lambda h: (0, 0)),
            pl.BlockSpec((1, D), lambda h: (0, 0)),
        ],
        out_specs=pl.BlockSpec((1, 8, BT, 8, 128), lambda h: (h, 0, 0, 0, 0)),
        out_shape=jax.ShapeDtypeStruct((H, 8, BT, 8, 128), jnp.float32),
    )(scr, gamma, beta)


def kernel(x, table, gamma, beta):
    x4 = x.astype(jnp.int32).T.reshape(HT, 8, BT, 128).transpose(0, 2, 1, 3)
    scr = _sc_gather(x4, table)
    out5 = _tc_ln(scr, gamma.reshape(1, D), beta.reshape(1, D))
    return out5.transpose(2, 4, 0, 1, 3).reshape(B, H, D)
